# Initial kernel scaffold; baseline (speedup 1.0000x reference)
import jax
import jax.numpy as jnp
from jax.experimental import pallas as pl


def kernel(x, sequence_lengths, table):
    raise NotImplementedError("write your pallas kernel here")


# speedup vs baseline: 1.6736x; 1.6736x over previous
import functools

import jax
import jax.numpy as jnp
from jax import lax
from jax.experimental import pallas as pl
from jax.experimental.pallas import tpu as pltpu
from jax.experimental.pallas import tpu_sc as plsc

EMBED_DIM = 32
BATCH = 16384
HIST = 20

NUM_CORES = 2
NUM_SUBCORES = 16
NUM_WORKERS = NUM_CORES * NUM_SUBCORES
SPW = BATCH // NUM_WORKERS
CHUNK = 64
NUM_CHUNKS = SPW // CHUNK
ROWS_PER_CHUNK = CHUNK * HIST
IDX_W = 128
DMAS_PER_CHUNK = ROWS_PER_CHUNK // IDX_W
IDX_ROWS = SPW * HIST // IDX_W
ACC_ROWS = NUM_SUBCORES * SPW + NUM_SUBCORES
ZCHUNK = 64


def _sc_body(table_hbm, x_hbm, dst_hbm, out_hbm, idx_v, dst_v, rows_v,
             zeros_v, acc_s, gsem, ssem):
    sid = lax.axis_index("s")
    cid = lax.axis_index("c")
    wid = sid * NUM_CORES + cid
    wbase = pl.multiple_of(wid * SPW, SPW)
    xrow = pl.multiple_of(wid * IDX_ROWS, IDX_ROWS)
    arow = pl.multiple_of(sid * SPW, SPW)

    pltpu.sync_copy(x_hbm.at[pl.ds(xrow, IDX_ROWS)], idx_v)
    pltpu.sync_copy(dst_hbm.at[pl.ds(xrow, IDX_ROWS)], dst_v)

    zero = jnp.zeros((16,), jnp.float32)
    for i in range(ZCHUNK):
        zeros_v[i, pl.ds(0, 16)] = zero
        zeros_v[i, pl.ds(16, 16)] = zero
    for z in range(SPW // ZCHUNK):
        pltpu.sync_copy(zeros_v, acc_s.at[pl.ds(arow + z * ZCHUNK, ZCHUNK)])

    for c in range(NUM_CHUNKS):
        gathers = [
            pltpu.async_copy(
                table_hbm.at[idx_v.at[c * DMAS_PER_CHUNK + j]],
                rows_v.at[pl.ds(j * IDX_W, IDX_W)],
                gsem,
            )
            for j in range(DMAS_PER_CHUNK)
        ]
        for cp in gathers:
            cp.wait()
        scatters = [
            pltpu.async_copy(
                rows_v.at[pl.ds(j * IDX_W, IDX_W)],
                acc_s.at[dst_v.at[c * DMAS_PER_CHUNK + j]],
                ssem,
                add=True,
            )
            for j in range(DMAS_PER_CHUNK)
        ]
        for cp in scatters:
            cp.wait()

    pltpu.sync_copy(acc_s.at[pl.ds(arow, SPW)],
                    out_hbm.at[pl.ds(wbase, SPW)])


def _scale_body(sums_ref, lens_ref, out_ref):
    lens = lens_ref[...].astype(jnp.float32)
    inv = 1.0 / jnp.maximum(lens, 1.0)
    out_ref[...] = sums_ref[...] * inv


@jax.jit
def _run(table, x2d, dst2d, lens):
    mesh = plsc.VectorSubcoreMesh(core_axis_name="c", subcore_axis_name="s")
    sums = functools.partial(
        pl.kernel,
        mesh=mesh,
        out_type=jax.ShapeDtypeStruct((BATCH, EMBED_DIM), jnp.float32),
        scratch_types=[
            pltpu.VMEM((IDX_ROWS, IDX_W), jnp.int32),
            pltpu.VMEM((IDX_ROWS, IDX_W), jnp.int32),
            pltpu.VMEM((ROWS_PER_CHUNK, EMBED_DIM), jnp.float32),
            pltpu.VMEM((ZCHUNK, EMBED_DIM), jnp.float32),
            pltpu.VMEM_SHARED((ACC_ROWS, EMBED_DIM), jnp.float32),
            pltpu.SemaphoreType.DMA,
            pltpu.SemaphoreType.DMA,
        ],
        compiler_params=pltpu.CompilerParams(use_tc_tiling_on_sc=False),
    )(_sc_body)(table, x2d, dst2d)

    return pl.pallas_call(
        _scale_body,
        out_shape=jax.ShapeDtypeStruct((BATCH, EMBED_DIM), jnp.float32),
    )(sums, lens.reshape(BATCH, 1))


def kernel(x, sequence_lengths, table):
    lens = sequence_lengths.astype(jnp.int32)
    xi = x.astype(jnp.int32)
    b = jnp.arange(BATCH, dtype=jnp.int32)
    slot = ((b // SPW) // NUM_CORES) * SPW + b % SPW
    trash = NUM_SUBCORES * SPW + (b // SPW) // NUM_CORES
    t = jnp.arange(HIST, dtype=jnp.int32)[None, :]
    valid = t < lens[:, None]
    dst = jnp.where(valid, slot[:, None], trash[:, None])
    x2d = xi.reshape(BATCH * HIST // IDX_W, IDX_W)
    dst2d = dst.reshape(BATCH * HIST // IDX_W, IDX_W)
    return _run(table, x2d, dst2d, lens)
